# Initial kernel scaffold; baseline (speedup 1.0000x reference)
import jax
import jax.numpy as jnp
from jax.experimental import pallas as pl


def kernel(nodes, to_neighs, neighs_r, u2e, w_r1_w, w_r1_b, w_r2_w, w_r2_b, att1_w, att1_b, att2_w, att2_b, att3_w, att3_b):
    raise NotImplementedError("write your pallas kernel here")


# speedup vs baseline: 11.5901x; 11.5901x over previous
import functools

import jax
import jax.numpy as jnp
from jax import lax
from jax.experimental import pallas as pl
from jax.experimental.pallas import tpu as pltpu
from jax.experimental.pallas import tpu_sc as plsc

N = 16384
K = 64
V = 1000000
D = 16

NC, NS = 2, 16
NW = NC * NS
CHUNK = 2048
GPC = CHUNK // 128
NEIGH_ITERS = (N * K) // (NW * CHUNK)
NODE_GPC = (N // NW) // 128

def _sc_gather_body(idx_hbm, nodes_hbm, table_hbm, e_out, u_out,
                    idx_v, rows_v, sem):
    wid = lax.axis_index("s") * NC + lax.axis_index("c")

    def body(g, carry):
        row0 = wid * (NEIGH_ITERS * GPC) + g * GPC
        pltpu.sync_copy(idx_hbm.at[pl.ds(row0, GPC)], idx_v)
        cps = [
            pltpu.async_copy(
                table_hbm.at[idx_v.at[j]],
                rows_v.at[pl.ds(j * 128, 128)], sem)
            for j in range(GPC)
        ]
        for cp in cps:
            cp.wait()
        pltpu.sync_copy(rows_v, e_out.at[pl.ds(row0 * 128, CHUNK)])
        return carry

    lax.fori_loop(0, NEIGH_ITERS, body, 0)

    nrow0 = wid * NODE_GPC
    pltpu.sync_copy(nodes_hbm.at[pl.ds(nrow0, NODE_GPC)],
                    idx_v.at[pl.ds(0, NODE_GPC)])
    cps = [
        pltpu.async_copy(
            table_hbm.at[idx_v.at[j]],
            rows_v.at[pl.ds(j * 128, 128)], sem)
        for j in range(NODE_GPC)
    ]
    for cp in cps:
        cp.wait()
    pltpu.sync_copy(rows_v.at[pl.ds(0, NODE_GPC * 128)],
                    u_out.at[pl.ds(wid * NODE_GPC * 128, NODE_GPC * 128)])


@functools.cache
def _sc_gather():
    mesh = plsc.VectorSubcoreMesh(
        core_axis_name="c", subcore_axis_name="s",
        num_cores=NC, num_subcores=NS)
    return pl.kernel(
        _sc_gather_body,
        out_type=[
            jax.ShapeDtypeStruct((N * K, D), jnp.float32),
            jax.ShapeDtypeStruct((N, D), jnp.float32),
        ],
        mesh=mesh,
        scratch_types=[
            pltpu.VMEM((GPC, 128), jnp.int32),
            pltpu.VMEM((CHUNK, D), jnp.float32),
            pltpu.SemaphoreType.DMA,
        ],
        compiler_params=pltpu.CompilerParams(use_tc_tiling_on_sc=False),
    )


BN = 128
ROWS = BN * K


def _tc_body(e_ref, r_ref, u_ref, w1a_ref, w1b_ref, b1_ref, w2_ref, b2_ref,
             a1o_ref, a1u_ref, ab1_ref, a2w_ref, ab2_ref, a3w_ref, ab3_ref,
             out_ref):
    f32 = jnp.float32
    e = e_ref[...]
    r = r_ref[...]
    x = jnp.dot(e, w1a_ref[...], preferred_element_type=f32)
    x = jnp.maximum(x + r * w1b_ref[...] + b1_ref[...], 0.0)
    o = jnp.dot(x, w2_ref[...], preferred_element_type=f32)
    o = jnp.maximum(o + b2_ref[...], 0.0)

    u = u_ref[...]
    uc = jnp.dot(u, a1u_ref[...], preferred_element_type=f32)
    a = jnp.dot(o, a1o_ref[...], preferred_element_type=f32)
    a = a.reshape(BN, K, D) + uc[:, None, :]
    a = jnp.maximum(a.reshape(ROWS, D) + ab1_ref[...], 0.0)
    a = jnp.dot(a, a2w_ref[...], preferred_element_type=f32)
    a = jnp.maximum(a + ab2_ref[...], 0.0)
    s = jnp.dot(a, a3w_ref[...], preferred_element_type=f32) + ab3_ref[...]

    s3 = s.reshape(BN, K, 1)
    m = jnp.max(s3, axis=1, keepdims=True)
    p = jnp.exp(s3 - m)
    den = jnp.sum(p, axis=1)
    num = jnp.sum(o.reshape(BN, K, D) * p, axis=1)
    out_ref[...] = num / den


def _tc_call(e_flat, r_col, urep, w1a, w1b, b1, w2, b2, a1o, a1u, ab1,
             a2w, ab2, a3w, ab3):
    nblk = N // BN
    full = lambda shape: pl.BlockSpec(shape, lambda i: (0, 0))
    return pl.pallas_call(
        _tc_body,
        grid=(nblk,),
        in_specs=[
            pl.BlockSpec((ROWS, D), lambda i: (i, 0)),
            pl.BlockSpec((ROWS, 1), lambda i: (i, 0)),
            pl.BlockSpec((BN, D), lambda i: (i, 0)),
            full((D, D)), full((1, D)), full((1, D)),
            full((D, D)), full((1, D)),
            full((D, D)), full((D, D)), full((1, D)),
            full((D, D)), full((1, D)),
            full((D, 1)), full((1, 1)),
        ],
        out_specs=pl.BlockSpec((BN, D), lambda i: (i, 0)),
        out_shape=jax.ShapeDtypeStruct((N, D), jnp.float32),
    )(e_flat, r_col, urep, w1a, w1b, b1, w2, b2, a1o, a1u, ab1,
      a2w, ab2, a3w, ab3)


def kernel(nodes, to_neighs, neighs_r, u2e,
           w_r1_w, w_r1_b, w_r2_w, w_r2_b,
           att1_w, att1_b, att2_w, att2_b, att3_w, att3_b):
    idx2d = to_neighs.reshape(N * K // 128, 128)
    nodes2d = nodes.reshape(N // 128, 128)
    e_flat, urep = _sc_gather()(idx2d, nodes2d, u2e)

    r_col = neighs_r.reshape(N * K, 1)
    w1a = w_r1_w[:D]
    w1b = w_r1_w[D:D + 1]
    b1 = w_r1_b.reshape(1, D)
    b2 = w_r2_b.reshape(1, D)
    a1o = att1_w[:D]
    a1u = att1_w[D:]
    ab1 = att1_b.reshape(1, D)
    ab2 = att2_b.reshape(1, D)
    ab3 = att3_b.reshape(1, 1)
    return _tc_call(e_flat, r_col, urep, w1a, w1b, b1, w_r2_w, b2,
                    a1o, a1u, ab1, att2_w, ab2, att3_w, ab3)


# speedup vs baseline: 26.8821x; 2.3194x over previous
import functools

import jax
import jax.numpy as jnp
from jax import lax
from jax.experimental import pallas as pl
from jax.experimental.pallas import tpu as pltpu
from jax.experimental.pallas import tpu_sc as plsc

N = 16384
K = 64
V = 1000000
D = 16

NC, NS = 2, 16
NW = NC * NS
CHUNK = 2048
GPC = CHUNK // 128
NEIGH_ITERS = (N * K) // (NW * CHUNK)
NODE_GPC = (N // NW) // 128

def _sc_gather_body(idx_hbm, nodes_hbm, table_hbm, e_out, u_out,
                    idx_v, rows_v, sem):
    wid = lax.axis_index("s") * NC + lax.axis_index("c")

    def body(g, carry):
        row0 = wid * (NEIGH_ITERS * GPC) + g * GPC
        pltpu.sync_copy(idx_hbm.at[pl.ds(row0, GPC)], idx_v)
        cps = [
            pltpu.async_copy(
                table_hbm.at[idx_v.at[j]],
                rows_v.at[pl.ds(j * 128, 128)], sem)
            for j in range(GPC)
        ]
        for cp in cps:
            cp.wait()
        pltpu.sync_copy(rows_v, e_out.at[pl.ds(row0 * 128, CHUNK)])
        return carry

    lax.fori_loop(0, NEIGH_ITERS, body, 0)

    nrow0 = wid * NODE_GPC
    pltpu.sync_copy(nodes_hbm.at[pl.ds(nrow0, NODE_GPC)],
                    idx_v.at[pl.ds(0, NODE_GPC)])
    cps = [
        pltpu.async_copy(
            table_hbm.at[idx_v.at[j]],
            rows_v.at[pl.ds(j * 128, 128)], sem)
        for j in range(NODE_GPC)
    ]
    for cp in cps:
        cp.wait()
    pltpu.sync_copy(rows_v.at[pl.ds(0, NODE_GPC * 128)],
                    u_out.at[pl.ds(wid * NODE_GPC * 128, NODE_GPC * 128)])


@functools.cache
def _sc_gather():
    mesh = plsc.VectorSubcoreMesh(
        core_axis_name="c", subcore_axis_name="s",
        num_cores=NC, num_subcores=NS)
    return pl.kernel(
        _sc_gather_body,
        out_type=[
            jax.ShapeDtypeStruct((N * K, D), jnp.float32),
            jax.ShapeDtypeStruct((N, D), jnp.float32),
        ],
        mesh=mesh,
        scratch_types=[
            pltpu.VMEM((GPC, 128), jnp.int32),
            pltpu.VMEM((CHUNK, D), jnp.float32),
            pltpu.SemaphoreType.DMA,
        ],
        compiler_params=pltpu.CompilerParams(use_tc_tiling_on_sc=False),
    )


BN = 128
P = 16
PL = P * D
PROWS = BN * K // P
RPN = K // P


def _tc_body(e_ref, r_ref, u_ref, bd1_ref, s1_ref, b1_ref, bd2_ref, b2_ref,
             bda1_ref, a1u_ref, ab1_ref, bda2_ref, ab2_ref, a3s_ref, ab3_ref,
             t16_ref, e16_ref, g16_ref, out_ref):
    f32 = jnp.float32
    dot = lambda x, y: jnp.dot(x, y, preferred_element_type=f32)
    e = e_ref[...]
    r = r_ref[...]
    x = jnp.maximum(dot(e, bd1_ref[...]) + dot(r, s1_ref[...]) + b1_ref[...],
                    0.0)
    o = jnp.maximum(dot(x, bd2_ref[...]) + b2_ref[...], 0.0)

    uc = dot(u_ref[...], a1u_ref[...])
    uct = dot(uc, t16_ref[...])
    ucb = jnp.broadcast_to(uct[:, None, :], (BN, RPN, PL)).reshape(PROWS, PL)
    a = jnp.maximum(dot(o, bda1_ref[...]) + ucb + ab1_ref[...], 0.0)
    a = jnp.maximum(dot(a, bda2_ref[...]) + ab2_ref[...], 0.0)
    s = dot(a, a3s_ref[...]) + ab3_ref[0, 0]

    s3 = s.reshape(BN, RPN, P)
    m = jnp.max(jnp.max(s3, axis=2, keepdims=True), axis=1, keepdims=True)
    p = jnp.exp(s3 - m)
    den = jnp.sum(jnp.sum(p, axis=2, keepdims=True), axis=1)
    wp = dot(p.reshape(PROWS, P), e16_ref[...])
    pr = dot(o * wp, g16_ref[...])
    num = jnp.sum(pr.reshape(BN, RPN, D), axis=1)
    out_ref[...] = num / den


def _tc_call(e_pack, r_pack, urep, bd1, s1, b1t, bd2, b2t, bda1, a1u, ab1t,
             bda2, ab2t, a3s, ab3, t16, e16, g16):
    nblk = N // BN
    full = lambda shape: pl.BlockSpec(shape, lambda i: (0, 0))
    return pl.pallas_call(
        _tc_body,
        grid=(nblk,),
        in_specs=[
            pl.BlockSpec((PROWS, PL), lambda i: (i, 0)),
            pl.BlockSpec((PROWS, P), lambda i: (i, 0)),
            pl.BlockSpec((BN, D), lambda i: (i, 0)),
            full((PL, PL)), full((P, PL)), full((1, PL)),
            full((PL, PL)), full((1, PL)),
            full((PL, PL)), full((D, D)), full((1, PL)),
            full((PL, PL)), full((1, PL)),
            full((PL, P)), full((1, 1)),
            full((D, PL)), full((P, PL)), full((PL, D)),
        ],
        out_specs=pl.BlockSpec((BN, D), lambda i: (i, 0)),
        out_shape=jax.ShapeDtypeStruct((N, D), jnp.float32),
    )(e_pack, r_pack, urep, bd1, s1, b1t, bd2, b2t, bda1, a1u, ab1t,
      bda2, ab2t, a3s, ab3, t16, e16, g16)


def kernel(nodes, to_neighs, neighs_r, u2e,
           w_r1_w, w_r1_b, w_r2_w, w_r2_b,
           att1_w, att1_b, att2_w, att2_b, att3_w, att3_b):
    idx2d = to_neighs.reshape(N * K // 128, 128)
    nodes2d = nodes.reshape(N // 128, 128)
    e_flat, urep = _sc_gather()(idx2d, nodes2d, u2e)

    e_pack = e_flat.reshape(N * K // P, PL)
    r_pack = neighs_r.reshape(N * K // P, P)
    eye = jnp.eye(P, dtype=jnp.float32)
    bd1 = jnp.kron(eye, w_r1_w[:D])
    s1 = jnp.kron(eye, w_r1_w[D:D + 1])
    b1t = jnp.tile(w_r1_b, P).reshape(1, PL)
    bd2 = jnp.kron(eye, w_r2_w)
    b2t = jnp.tile(w_r2_b, P).reshape(1, PL)
    bda1 = jnp.kron(eye, att1_w[:D])
    a1u = att1_w[D:]
    ab1t = jnp.tile(att1_b, P).reshape(1, PL)
    bda2 = jnp.kron(eye, att2_w)
    ab2t = jnp.tile(att2_b, P).reshape(1, PL)
    a3s = jnp.kron(eye, att3_w)
    ab3 = att3_b.reshape(1, 1)
    t16 = jnp.tile(jnp.eye(D, dtype=jnp.float32), (1, P))
    e16 = jnp.repeat(eye, D, axis=1)
    g16 = t16.T
    return _tc_call(e_pack, r_pack, urep, bd1, s1, b1t, bd2, b2t, bda1, a1u,
                    ab1t, bda2, ab2t, a3s, ab3, t16, e16, g16)


# speedup vs baseline: 29.0368x; 1.0802x over previous
import functools

import jax
import jax.numpy as jnp
from jax import lax
from jax.experimental import pallas as pl
from jax.experimental.pallas import tpu as pltpu
from jax.experimental.pallas import tpu_sc as plsc

N = 16384
K = 64
V = 1000000
D = 16

NC, NS = 2, 16
NW = NC * NS
CHUNK = 2048
GPC = CHUNK // 128
NEIGH_ITERS = (N * K) // (NW * CHUNK)
NODE_GPC = (N // NW) // 128

def _sc_gather_body(idx_hbm, nodes_hbm, table_hbm, e_out, u_out,
                    idx_v, rows_v, pk_v, sem):
    wid = lax.axis_index("s") * NC + lax.axis_index("c")
    table = table_hbm
    prows_per_chunk = CHUNK * D // 128

    def body(g, carry):
        row0 = wid * (NEIGH_ITERS * GPC) + g * GPC
        pltpu.sync_copy(idx_hbm.at[pl.ds(row0, GPC)], idx_v)
        cps = [
            pltpu.async_copy(
                table.at[idx_v.at[j]],
                rows_v.at[pl.ds(j * 128, 128)], sem)
            for j in range(GPC)
        ]
        for cp in cps:
            cp.wait()

        def repack(pr, c2):
            for c in range(8):
                pk_v[pr, pl.ds(c * D, D)] = rows_v[pr * 8 + c, :]
            return c2

        lax.fori_loop(0, prows_per_chunk, repack, 0)
        pltpu.sync_copy(pk_v,
                        e_out.at[pl.ds(row0 * 16, prows_per_chunk)])
        return carry

    lax.fori_loop(0, NEIGH_ITERS, body, 0)

    nrow0 = wid * NODE_GPC
    pltpu.sync_copy(nodes_hbm.at[pl.ds(nrow0, NODE_GPC)],
                    idx_v.at[pl.ds(0, NODE_GPC)])
    cps = [
        pltpu.async_copy(
            table.at[idx_v.at[j]],
            rows_v.at[pl.ds(j * 128, 128)], sem)
        for j in range(NODE_GPC)
    ]
    for cp in cps:
        cp.wait()
    pltpu.sync_copy(rows_v.at[pl.ds(0, NODE_GPC * 128)],
                    u_out.at[pl.ds(wid * NODE_GPC * 128, NODE_GPC * 128)])


@functools.cache
def _sc_gather():
    mesh = plsc.VectorSubcoreMesh(
        core_axis_name="c", subcore_axis_name="s",
        num_cores=NC, num_subcores=NS)
    return pl.kernel(
        _sc_gather_body,
        out_type=[
            jax.ShapeDtypeStruct((N * K * D // 128, 128), jnp.float32),
            jax.ShapeDtypeStruct((N, D), jnp.float32),
        ],
        mesh=mesh,
        scratch_types=[
            pltpu.VMEM((GPC, 128), jnp.int32),
            pltpu.VMEM((CHUNK, D), jnp.float32),
            pltpu.VMEM((CHUNK * D // 128, 128), jnp.float32),
            pltpu.SemaphoreType.DMA,
        ],
        compiler_params=pltpu.CompilerParams(use_tc_tiling_on_sc=False),
    )


BN = 128
P = 8
PL = P * D
PROWS = BN * K // P
RPN = K // P


def _tc_body(e_ref, r_ref, u_ref, bd1_ref, s1_ref, b1_ref, bd2_ref, b2_ref,
             bda1_ref, a1u_ref, ab1_ref, bda2_ref, ab2_ref, a3s_ref, ab3_ref,
             t16_ref, e16_ref, g16_ref, out_ref):
    f32 = jnp.float32
    dot = lambda x, y: jnp.dot(x, y, preferred_element_type=f32)
    e = e_ref[...]
    r = r_ref[...]
    x = jnp.maximum(dot(e, bd1_ref[...]) + dot(r, s1_ref[...]) + b1_ref[...],
                    0.0)
    o = jnp.maximum(dot(x, bd2_ref[...]) + b2_ref[...], 0.0)

    uc = dot(u_ref[...], a1u_ref[...])
    uct = dot(uc, t16_ref[...])
    ucb = jnp.broadcast_to(uct[:, None, :], (BN, RPN, PL)).reshape(PROWS, PL)
    a = jnp.maximum(dot(o, bda1_ref[...]) + ucb + ab1_ref[...], 0.0)
    a = jnp.maximum(dot(a, bda2_ref[...]) + ab2_ref[...], 0.0)
    s = dot(a, a3s_ref[...]) + ab3_ref[0, 0]

    s3 = s.reshape(BN, RPN, P)
    m = jnp.max(jnp.max(s3, axis=2, keepdims=True), axis=1, keepdims=True)
    p = jnp.exp(s3 - m)
    den = jnp.sum(jnp.sum(p, axis=2, keepdims=True), axis=1)
    wp = dot(p.reshape(PROWS, P), e16_ref[...])
    pr = dot(o * wp, g16_ref[...])
    num = jnp.sum(pr.reshape(BN, RPN, D), axis=1)
    out_ref[...] = num / den


def _tc_call(e_pack, r_pack, urep, bd1, s1, b1t, bd2, b2t, bda1, a1u, ab1t,
             bda2, ab2t, a3s, ab3, t16, e16, g16):
    nblk = N // BN
    full = lambda shape: pl.BlockSpec(shape, lambda i: (0, 0))
    return pl.pallas_call(
        _tc_body,
        grid=(nblk,),
        in_specs=[
            pl.BlockSpec((PROWS, PL), lambda i: (i, 0)),
            pl.BlockSpec((PROWS, P), lambda i: (i, 0)),
            pl.BlockSpec((BN, D), lambda i: (i, 0)),
            full((PL, PL)), full((P, PL)), full((1, PL)),
            full((PL, PL)), full((1, PL)),
            full((PL, PL)), full((D, D)), full((1, PL)),
            full((PL, PL)), full((1, PL)),
            full((PL, P)), full((1, 1)),
            full((D, PL)), full((P, PL)), full((PL, D)),
        ],
        out_specs=pl.BlockSpec((BN, D), lambda i: (i, 0)),
        out_shape=jax.ShapeDtypeStruct((N, D), jnp.float32),
    )(e_pack, r_pack, urep, bd1, s1, b1t, bd2, b2t, bda1, a1u, ab1t,
      bda2, ab2t, a3s, ab3, t16, e16, g16)


def kernel(nodes, to_neighs, neighs_r, u2e,
           w_r1_w, w_r1_b, w_r2_w, w_r2_b,
           att1_w, att1_b, att2_w, att2_b, att3_w, att3_b):
    idx2d = to_neighs.reshape(N * K // 128, 128)
    nodes2d = nodes.reshape(N // 128, 128)
    e_pack, urep = _sc_gather()(idx2d, nodes2d, u2e)

    r_pack = neighs_r.reshape(N * K // P, P)
    eye = jnp.eye(P, dtype=jnp.float32)
    bd1 = jnp.kron(eye, w_r1_w[:D])
    s1 = jnp.kron(eye, w_r1_w[D:D + 1])
    b1t = jnp.tile(w_r1_b, P).reshape(1, PL)
    bd2 = jnp.kron(eye, w_r2_w)
    b2t = jnp.tile(w_r2_b, P).reshape(1, PL)
    bda1 = jnp.kron(eye, att1_w[:D])
    a1u = att1_w[D:]
    ab1t = jnp.tile(att1_b, P).reshape(1, PL)
    bda2 = jnp.kron(eye, att2_w)
    ab2t = jnp.tile(att2_b, P).reshape(1, PL)
    a3s = jnp.kron(eye, att3_w)
    ab3 = att3_b.reshape(1, 1)
    t16 = jnp.tile(jnp.eye(D, dtype=jnp.float32), (1, P))
    e16 = jnp.repeat(eye, D, axis=1)
    g16 = t16.T
    return _tc_call(e_pack, r_pack, urep, bd1, s1, b1t, bd2, b2t, bda1, a1u,
                    ab1t, bda2, ab2t, a3s, ab3, t16, e16, g16)


# speedup vs baseline: 35.5837x; 1.2255x over previous
import functools

import jax
import jax.numpy as jnp
from jax import lax
from jax.experimental import pallas as pl
from jax.experimental.pallas import tpu as pltpu
from jax.experimental.pallas import tpu_sc as plsc

N = 16384
K = 64
V = 1000000
D = 16

NC, NS = 2, 16
NW = NC * NS
CHUNK = 2048
GPC = CHUNK // 128
NEIGH_ITERS = (N * K) // (NW * CHUNK)
NODE_GPC = (N // NW) // 128

def _sc_gather_body(idx_hbm, nodes_hbm, table_hbm, e_out, u_out,
                    idx_v, rows_v, pk_v, sem):
    wid = lax.axis_index("s") * NC + lax.axis_index("c")
    table = table_hbm
    prows_per_chunk = CHUNK * D // 128

    def body(g, carry):
        row0 = wid * (NEIGH_ITERS * GPC) + g * GPC
        pltpu.sync_copy(idx_hbm.at[pl.ds(row0, GPC)], idx_v)
        cps = [
            pltpu.async_copy(
                table.at[idx_v.at[j]],
                rows_v.at[pl.ds(j * 128, 128)], sem)
            for j in range(GPC)
        ]
        for cp in cps:
            cp.wait()

        def repack(pr, c2):
            for c in range(8):
                pk_v[pr, pl.ds(c * D, D)] = rows_v[pr * 8 + c, :]
            return c2

        lax.fori_loop(0, prows_per_chunk, repack, 0)
        pltpu.sync_copy(pk_v,
                        e_out.at[pl.ds(row0 * 16, prows_per_chunk)])
        return carry

    lax.fori_loop(0, NEIGH_ITERS, body, 0)

    nrow0 = wid * NODE_GPC
    pltpu.sync_copy(nodes_hbm.at[pl.ds(nrow0, NODE_GPC)],
                    idx_v.at[pl.ds(0, NODE_GPC)])
    cps = [
        pltpu.async_copy(
            table.at[idx_v.at[j]],
            rows_v.at[pl.ds(j * 128, 128)], sem)
        for j in range(NODE_GPC)
    ]
    for cp in cps:
        cp.wait()
    pltpu.sync_copy(rows_v.at[pl.ds(0, NODE_GPC * 128)],
                    u_out.at[pl.ds(wid * NODE_GPC * 128, NODE_GPC * 128)])


@functools.cache
def _sc_gather():
    mesh = plsc.VectorSubcoreMesh(
        core_axis_name="c", subcore_axis_name="s",
        num_cores=NC, num_subcores=NS)
    return pl.kernel(
        _sc_gather_body,
        out_type=[
            jax.ShapeDtypeStruct((N * K * D // 128, 128), jnp.float32),
            jax.ShapeDtypeStruct((N, D), jnp.float32),
        ],
        mesh=mesh,
        scratch_types=[
            pltpu.VMEM((GPC, 128), jnp.int32),
            pltpu.VMEM((CHUNK, D), jnp.float32),
            pltpu.VMEM((CHUNK * D // 128, 128), jnp.float32),
            pltpu.SemaphoreType.DMA,
        ],
        compiler_params=pltpu.CompilerParams(use_tc_tiling_on_sc=False),
    )


TBLK = 8192


def _repack_body(t_ref, o_ref):
    t = jnp.transpose(t_ref[...])
    t3 = t.reshape(TBLK // 8, 8, D)
    o_ref[...] = jnp.concatenate([t3[:, c, :] for c in range(8)], axis=1)


def _repack_table(u2e_t):
    return pl.pallas_call(
        _repack_body,
        grid=(pl.cdiv(V, TBLK),),
        in_specs=[pl.BlockSpec((D, TBLK), lambda i: (0, i))],
        out_specs=pl.BlockSpec((TBLK // 8, 128), lambda i: (i, 0)),
        out_shape=jax.ShapeDtypeStruct((V // 8, 128), jnp.float32),
    )(u2e_t)


BN = 128
P = 8
PL = P * D
PROWS = BN * K // P
RPN = K // P


def _tc_body(e_ref, r_ref, u_ref, bd1_ref, s1_ref, b1_ref, bd2_ref, b2_ref,
             bda1_ref, a1u_ref, ab1_ref, bda2_ref, ab2_ref, a3s_ref, ab3_ref,
             t16_ref, e16_ref, g16_ref, out_ref):
    f32 = jnp.float32
    dot = lambda x, y: jnp.dot(x, y, preferred_element_type=f32)
    e = e_ref[...]
    rt = r_ref[...]
    z = jax.lax.dot_general(rt, s1_ref[...], (((0,), (0,)), ((), ())),
                            preferred_element_type=f32)
    rterm = z.reshape(PROWS, PL)
    x = jnp.maximum(dot(e, bd1_ref[...]) + rterm + b1_ref[...], 0.0)
    o = jnp.maximum(dot(x, bd2_ref[...]) + b2_ref[...], 0.0)

    uc = dot(u_ref[...], a1u_ref[...])
    uct = dot(uc, t16_ref[...])
    ucb = jnp.broadcast_to(uct[:, None, :], (BN, RPN, PL)).reshape(PROWS, PL)
    a = jnp.maximum(dot(o, bda1_ref[...]) + ucb + ab1_ref[...], 0.0)
    a = jnp.maximum(dot(a, bda2_ref[...]) + ab2_ref[...], 0.0)
    s = dot(a, a3s_ref[...]) + ab3_ref[0, 0]

    s3 = s.reshape(BN, RPN, P)
    m = jnp.max(jnp.max(s3, axis=2, keepdims=True), axis=1, keepdims=True)
    p = jnp.exp(s3 - m)
    den = jnp.sum(jnp.sum(p, axis=2, keepdims=True), axis=1)
    wp = dot(p.reshape(PROWS, P), e16_ref[...])
    pr = dot(o * wp, g16_ref[...])
    num = jnp.sum(pr.reshape(BN, RPN, D), axis=1)
    out_ref[...] = num / den


def _tc_call(e_pack, r_pack, urep, bd1, s1, b1t, bd2, b2t, bda1, a1u, ab1t,
             bda2, ab2t, a3s, ab3, t16, e16, g16):
    nblk = N // BN
    full = lambda shape: pl.BlockSpec(shape, lambda i: (0, 0))
    return pl.pallas_call(
        _tc_body,
        grid=(nblk,),
        in_specs=[
            pl.BlockSpec((PROWS, PL), lambda i: (i, 0)),
            pl.BlockSpec((K, BN), lambda i: (0, i)),
            pl.BlockSpec((BN, D), lambda i: (i, 0)),
            full((PL, PL)), full((K, K * D)), full((1, PL)),
            full((PL, PL)), full((1, PL)),
            full((PL, PL)), full((D, D)), full((1, PL)),
            full((PL, PL)), full((1, PL)),
            full((PL, P)), full((1, 1)),
            full((D, PL)), full((P, PL)), full((PL, D)),
        ],
        out_specs=pl.BlockSpec((BN, D), lambda i: (i, 0)),
        out_shape=jax.ShapeDtypeStruct((N, D), jnp.float32),
    )(e_pack, r_pack, urep, bd1, s1, b1t, bd2, b2t, bda1, a1u, ab1t,
      bda2, ab2t, a3s, ab3, t16, e16, g16)


def kernel(nodes, to_neighs, neighs_r, u2e,
           w_r1_w, w_r1_b, w_r2_w, w_r2_b,
           att1_w, att1_b, att2_w, att2_b, att3_w, att3_b):
    idx2d = to_neighs.reshape(N * K // 128, 128)
    nodes2d = nodes.reshape(N // 128, 128)
    tbl = _repack_table(u2e.T).reshape(V, D)
    e_pack, urep = _sc_gather()(idx2d, nodes2d, tbl)

    r_t = neighs_r.T
    eye = jnp.eye(P, dtype=jnp.float32)
    bd1 = jnp.kron(eye, w_r1_w[:D])
    s1 = jnp.kron(jnp.eye(K, dtype=jnp.float32), w_r1_w[D:D + 1])
    b1t = jnp.tile(w_r1_b, P).reshape(1, PL)
    bd2 = jnp.kron(eye, w_r2_w)
    b2t = jnp.tile(w_r2_b, P).reshape(1, PL)
    bda1 = jnp.kron(eye, att1_w[:D])
    a1u = att1_w[D:]
    ab1t = jnp.tile(att1_b, P).reshape(1, PL)
    bda2 = jnp.kron(eye, att2_w)
    ab2t = jnp.tile(att2_b, P).reshape(1, PL)
    a3s = jnp.kron(eye, att3_w)
    ab3 = att3_b.reshape(1, 1)
    t16 = jnp.tile(jnp.eye(D, dtype=jnp.float32), (1, P))
    e16 = jnp.repeat(eye, D, axis=1)
    g16 = t16.T
    return _tc_call(e_pack, r_t, urep, bd1, s1, b1t, bd2, b2t, bda1, a1u,
                    ab1t, bda2, ab2t, a3s, ab3, t16, e16, g16)


# speedup vs baseline: 52.4244x; 1.4733x over previous
import functools

import jax
import jax.numpy as jnp
from jax import lax
from jax.experimental import pallas as pl
from jax.experimental.pallas import tpu as pltpu
from jax.experimental.pallas import tpu_sc as plsc

N = 16384
K = 64
V = 1000000
D = 16

NC, NS = 2, 16
NW = NC * NS
CHUNK = 2048
GPC = CHUNK // 128
NEIGH_ITERS = (N * K) // (NW * CHUNK)
NODE_GPC = (N // NW) // 128

def _sc_gather_body(idx_hbm, nodes_hbm, table_hbm, e_out, u_out,
                    idx_v, rows_v, pk_v, sem):
    wid = lax.axis_index("s") * NC + lax.axis_index("c")
    table = table_hbm
    prows_per_chunk = CHUNK * D // 128

    def body(g, carry):
        row0 = wid * (NEIGH_ITERS * GPC) + g * GPC
        pltpu.sync_copy(idx_hbm.at[pl.ds(row0, GPC)], idx_v)
        cps = [
            pltpu.async_copy(
                table.at[idx_v.at[j]],
                rows_v.at[pl.ds(j * 128, 128)], sem)
            for j in range(GPC)
        ]
        for cp in cps:
            cp.wait()

        def repack(pr, c2):
            for c in range(8):
                pk_v[pr, pl.ds(c * D, D)] = rows_v[pr * 8 + c, :]
            return c2

        lax.fori_loop(0, prows_per_chunk, repack, 0)
        pltpu.sync_copy(pk_v,
                        e_out.at[pl.ds(row0 * 16, prows_per_chunk)])
        return carry

    lax.fori_loop(0, NEIGH_ITERS, body, 0)

    nrow0 = wid * NODE_GPC
    pltpu.sync_copy(nodes_hbm.at[pl.ds(nrow0, NODE_GPC)],
                    idx_v.at[pl.ds(0, NODE_GPC)])
    cps = [
        pltpu.async_copy(
            table.at[idx_v.at[j]],
            rows_v.at[pl.ds(j * 128, 128)], sem)
        for j in range(NODE_GPC)
    ]
    for cp in cps:
        cp.wait()
    pltpu.sync_copy(rows_v.at[pl.ds(0, NODE_GPC * 128)],
                    u_out.at[pl.ds(wid * NODE_GPC * 128, NODE_GPC * 128)])


@functools.cache
def _sc_gather():
    mesh = plsc.VectorSubcoreMesh(
        core_axis_name="c", subcore_axis_name="s",
        num_cores=NC, num_subcores=NS)
    return pl.kernel(
        _sc_gather_body,
        out_type=[
            jax.ShapeDtypeStruct((N * K * D // 128, 128), jnp.float32),
            jax.ShapeDtypeStruct((N, D), jnp.float32),
        ],
        mesh=mesh,
        scratch_types=[
            pltpu.VMEM((GPC, 128), jnp.int32),
            pltpu.VMEM((CHUNK, D), jnp.float32),
            pltpu.VMEM((CHUNK * D // 128, 128), jnp.float32),
            pltpu.SemaphoreType.DMA,
        ],
        compiler_params=pltpu.CompilerParams(use_tc_tiling_on_sc=False),
    )


TBLK = 8192
PADV = ((V + TBLK - 1) // TBLK) * TBLK


def _repack_body(t_ref, o_ref):
    t = t_ref[...]
    stk = jnp.concatenate(
        [t[:, c * (TBLK // 8):(c + 1) * (TBLK // 8)] for c in range(8)],
        axis=0)
    o_ref[...] = jnp.transpose(stk)


def _repack_table(u2e_t):
    return pl.pallas_call(
        _repack_body,
        grid=(pl.cdiv(V, TBLK),),
        in_specs=[pl.BlockSpec((D, TBLK), lambda i: (0, i))],
        out_specs=pl.BlockSpec((TBLK // 8, 128), lambda i: (i, 0)),
        out_shape=jax.ShapeDtypeStruct((PADV // 8, 128), jnp.float32),
    )(u2e_t)


BN = 128
P = 8
PL = P * D
PROWS = BN * K // P
RPN = K // P


def _tc_body(e_ref, r_ref, u_ref, bd1_ref, s1_ref, b1_ref, bd2_ref, b2_ref,
             bda1_ref, a1u_ref, ab1_ref, bda2_ref, ab2_ref, a3s_ref, ab3_ref,
             t16_ref, e16_ref, g16_ref, out_ref):
    f32 = jnp.float32
    dot = lambda x, y: jnp.dot(x, y, preferred_element_type=f32)
    e = e_ref[...]
    rt = r_ref[...]
    z = jax.lax.dot_general(rt, s1_ref[...], (((0,), (0,)), ((), ())),
                            preferred_element_type=f32)
    rterm = z.reshape(PROWS, PL)
    x = jnp.maximum(dot(e, bd1_ref[...]) + rterm + b1_ref[...], 0.0)
    o = jnp.maximum(dot(x, bd2_ref[...]) + b2_ref[...], 0.0)

    uc = dot(u_ref[...], a1u_ref[...])
    uct = dot(uc, t16_ref[...])
    ucb = jnp.broadcast_to(uct[:, None, :], (BN, RPN, PL)).reshape(PROWS, PL)
    a = jnp.maximum(dot(o, bda1_ref[...]) + ucb + ab1_ref[...], 0.0)
    a = jnp.maximum(dot(a, bda2_ref[...]) + ab2_ref[...], 0.0)
    s = dot(a, a3s_ref[...]) + ab3_ref[0, 0]

    s3 = s.reshape(BN, RPN, P)
    m = jnp.max(jnp.max(s3, axis=2, keepdims=True), axis=1, keepdims=True)
    p = jnp.exp(s3 - m)
    den = jnp.sum(jnp.sum(p, axis=2, keepdims=True), axis=1)
    wp = dot(p.reshape(PROWS, P), e16_ref[...])
    pr = dot(o * wp, g16_ref[...])
    num = jnp.sum(pr.reshape(BN, RPN, D), axis=1)
    out_ref[...] = num / den


def _tc_call(e_pack, r_pack, urep, bd1, s1, b1t, bd2, b2t, bda1, a1u, ab1t,
             bda2, ab2t, a3s, ab3, t16, e16, g16):
    nblk = N // BN
    full = lambda shape: pl.BlockSpec(shape, lambda i: (0, 0))
    return pl.pallas_call(
        _tc_body,
        grid=(nblk,),
        in_specs=[
            pl.BlockSpec((PROWS, PL), lambda i: (i, 0)),
            pl.BlockSpec((K, BN), lambda i: (0, i)),
            pl.BlockSpec((BN, D), lambda i: (i, 0)),
            full((PL, PL)), full((K, K * D)), full((1, PL)),
            full((PL, PL)), full((1, PL)),
            full((PL, PL)), full((D, D)), full((1, PL)),
            full((PL, PL)), full((1, PL)),
            full((PL, P)), full((1, 1)),
            full((D, PL)), full((P, PL)), full((PL, D)),
        ],
        out_specs=pl.BlockSpec((BN, D), lambda i: (i, 0)),
        out_shape=jax.ShapeDtypeStruct((N, D), jnp.float32),
    )(e_pack, r_pack, urep, bd1, s1, b1t, bd2, b2t, bda1, a1u, ab1t,
      bda2, ab2t, a3s, ab3, t16, e16, g16)


def kernel(nodes, to_neighs, neighs_r, u2e,
           w_r1_w, w_r1_b, w_r2_w, w_r2_b,
           att1_w, att1_b, att2_w, att2_b, att3_w, att3_b):
    def _perm(v):
        t = v & (TBLK - 1)
        return (v - t) | ((t & (TBLK // 8 - 1)) << 3) | (t >> 10)

    idx2d = _perm(to_neighs).reshape(N * K // 128, 128)
    nodes2d = _perm(nodes).reshape(N // 128, 128)
    tbl = _repack_table(u2e.T).reshape(PADV, D)
    e_pack, urep = _sc_gather()(idx2d, nodes2d, tbl)

    r_t = neighs_r.T
    eye = jnp.eye(P, dtype=jnp.float32)
    bd1 = jnp.kron(eye, w_r1_w[:D])
    s1 = jnp.kron(jnp.eye(K, dtype=jnp.float32), w_r1_w[D:D + 1])
    b1t = jnp.tile(w_r1_b, P).reshape(1, PL)
    bd2 = jnp.kron(eye, w_r2_w)
    b2t = jnp.tile(w_r2_b, P).reshape(1, PL)
    bda1 = jnp.kron(eye, att1_w[:D])
    a1u = att1_w[D:]
    ab1t = jnp.tile(att1_b, P).reshape(1, PL)
    bda2 = jnp.kron(eye, att2_w)
    ab2t = jnp.tile(att2_b, P).reshape(1, PL)
    a3s = jnp.kron(eye, att3_w)
    ab3 = att3_b.reshape(1, 1)
    t16 = jnp.tile(jnp.eye(D, dtype=jnp.float32), (1, P))
    e16 = jnp.repeat(eye, D, axis=1)
    g16 = t16.T
    return _tc_call(e_pack, r_t, urep, bd1, s1, b1t, bd2, b2t, bda1, a1u,
                    ab1t, bda2, ab2t, a3s, ab3, t16, e16, g16)
